# Initial kernel scaffold; baseline (speedup 1.0000x reference)
import jax
import jax.numpy as jnp
from jax.experimental import pallas as pl


def kernel(x_paper, x_author, ei_pa, ei_ap, ei_pp, l1_pw_p, l1_pb_p, l1_pw_a, l1_pb_a, l1_as_pa, l1_ad_pa, l1_as_ap, l1_ad_ap, l1_as_pp, l1_ad_pp, l1_q, l1_kw, l1_kb, l2_pw_p, l2_pb_p, l2_pw_a, l2_pb_a, l2_as_pa, l2_ad_pa, l2_as_ap, l2_ad_ap, l2_as_pp, l2_ad_pp, l2_q, l2_kw, l2_kb, lin_w, lin_b):
    raise NotImplementedError("write your pallas kernel here")


# speedup vs baseline: 36.9195x; 36.9195x over previous
import functools

import jax
import jax.numpy as jnp
from jax import lax
from jax.experimental import pallas as pl
from jax.experimental.pallas import tpu as pltpu
from jax.experimental.pallas import tpu_sc as plsc

N = 10000
E = 320000
IN_DIM = 128
HIDDEN = 64
HEADS = 8
DH = 8
NUM_CLASSES = 8

XW = 128
NW = 32
CHUNK = 128
NCHUNK = 80
EPT = CHUNK * NCHUNK
EPAD = EPT * NW
NPAD = 10112
RPT = NPAD // 16


def _c16(v):
    return jnp.full((16,), v, jnp.int32)


def _conv_body(xext, adst, ridx, cidx, out, idxr_v, idxc_v, xbuf, dbuf,
               accum):
    cid = lax.axis_index("c")
    sid = lax.axis_index("s")
    wid = sid * 2 + cid

    zero16 = jnp.zeros((16,), jnp.float32)

    def zrow(r, carry):
        for c in range(XW // 16):
            xbuf[r, pl.ds(c * 16, 16)] = zero16
        return carry

    lax.fori_loop(0, CHUNK, zrow, 0)
    base = sid * RPT

    def zstrip(t, carry):
        pltpu.sync_copy(xbuf.at[pl.ds(0, 8)], accum.at[pl.ds(base + t * 8, 8)])
        return carry

    lax.fori_loop(0, RPT // 8, zstrip, 0)

    plsc.subcore_barrier()

    def chunk(j, carry):
        row = wid * NCHUNK + j
        pltpu.sync_copy(ridx.at[row], idxr_v)
        pltpu.sync_copy(cidx.at[row], idxc_v)
        pltpu.sync_copy(xext.at[idxr_v], xbuf)
        pltpu.sync_copy(adst.at[idxc_v], dbuf)

        def grp(g, c2):
            for k in range(16):
                e = g * 16 + k
                adv = dbuf[e, pl.ds(0, 16)]
                asv = xbuf[e, pl.ds(HIDDEN, 16)]
                al = asv + adv
                al = jnp.maximum(al, al * 0.2)
                w = jnp.exp(al)
                xbuf[e, pl.ds(HIDDEN, 16)] = w
                for jj in range(HIDDEN // 16):
                    xv = xbuf[e, pl.ds(16 * jj, 16)]
                    xbuf[e, pl.ds(16 * jj, 16)] = xv * w
            return c2

        lax.fori_loop(0, CHUNK // 16, grp, 0)
        pltpu.sync_copy(xbuf, accum.at[idxc_v], add=True)
        return carry

    lax.fori_loop(0, NCHUNK, chunk, 0)
    plsc.subcore_barrier()

    def dstrip(t, carry):
        pltpu.sync_copy(accum.at[pl.ds(base + t * 8, 8)], xbuf.at[pl.ds(0, 8)])
        pltpu.sync_copy(xbuf.at[pl.ds(0, 8)],
                        out.at[cid, pl.ds(base + t * 8, 8)])
        return carry

    lax.fori_loop(0, RPT // 8, dstrip, 0)


@functools.cache
def _get_conv():
    return pl.kernel(
        _conv_body,
        out_type=jax.ShapeDtypeStruct((2, NPAD, XW), jnp.float32),
        mesh=plsc.VectorSubcoreMesh(core_axis_name="c", subcore_axis_name="s",
                                    num_cores=2, num_subcores=16),
        compiler_params=pltpu.CompilerParams(needs_layout_passes=False),
        scratch_types=[
            pltpu.VMEM((CHUNK,), jnp.int32),
            pltpu.VMEM((CHUNK,), jnp.int32),
            pltpu.VMEM((CHUNK, XW), jnp.float32),
            pltpu.VMEM((CHUNK, XW), jnp.float32),
            pltpu.VMEM_SHARED((NPAD, XW), jnp.float32),
        ],
    )


def _prep1_body(xp, xa, w1, b1, w2, b2, w3, b3, w4, b4, w5, b5, w6, b6,
                o1, o2, o3, o4, o5, o6):
    xpv = xp[...]
    xav = xa[...]
    o1[...] = jnp.dot(xpv, w1[...], preferred_element_type=jnp.float32) + b1[...]
    o2[...] = jnp.dot(xav, w2[...], preferred_element_type=jnp.float32) + b2[...]
    o3[...] = jnp.dot(xav, w3[...], preferred_element_type=jnp.float32) + b3[...]
    o4[...] = jnp.dot(xpv, w4[...], preferred_element_type=jnp.float32) + b4[...]
    o5[...] = jnp.dot(xpv, w5[...], preferred_element_type=jnp.float32) + b5[...]
    o6[...] = jnp.dot(xpv, w6[...], preferred_element_type=jnp.float32) + b6[...]


def _norm(acc0, acc1, s1, s2):
    acc = acc0[...] + acc1[...]
    num = jnp.dot(acc, s1, preferred_element_type=jnp.float32)
    den = jnp.dot(acc, s2, preferred_element_type=jnp.float32)
    return jnp.maximum(num / (den + 1e-16), 0.0)


def _sem_attn(o1, o2, kw, kb, q):
    t1 = jnp.tanh(jnp.dot(o1, kw, preferred_element_type=jnp.float32) + kb)
    t2 = jnp.tanh(jnp.dot(o2, kw, preferred_element_type=jnp.float32) + kb)
    m1 = jnp.sum(t1, axis=0, keepdims=True) * (1.0 / N)
    m2 = jnp.sum(t2, axis=0, keepdims=True) * (1.0 / N)
    sc1 = jnp.sum(q * m1, axis=1, keepdims=True)
    sc2 = jnp.sum(q * m2, axis=1, keepdims=True)
    mx = jnp.maximum(sc1, sc2)
    e1 = jnp.exp(sc1 - mx)
    e2 = jnp.exp(sc2 - mx)
    inv = 1.0 / (e1 + e2)
    return (e1 * inv) * o1 + (e2 * inv) * o2


def _norm3_body(apa0, apa1, aap0, aap1, app0, app1, s1r, s2r,
                o_a_ref, o_p1_ref, o_p2_ref):
    s1 = s1r[...]
    s2 = s2r[...]
    o_a_ref[...] = _norm(apa0, apa1, s1, s2)
    o_p1_ref[...] = _norm(aap0, aap1, s1, s2)
    o_p2_ref[...] = _norm(app0, app1, s1, s2)


def _score2_body(o1, o2, kw, kb, q, sc1_ref, sc2_ref):
    t1 = jnp.tanh(jnp.dot(o1[...], kw[...], preferred_element_type=jnp.float32)
                  + kb[...])
    t2 = jnp.tanh(jnp.dot(o2[...], kw[...], preferred_element_type=jnp.float32)
                  + kb[...])
    m1 = jnp.sum(t1, axis=0, keepdims=True) * (1.0 / N)
    m2 = jnp.sum(t2, axis=0, keepdims=True) * (1.0 / N)
    sc1_ref[...] = jnp.sum(q[...] * m1, axis=1, keepdims=True)
    sc2_ref[...] = jnp.sum(q[...] * m2, axis=1, keepdims=True)


def _combine(o1, o2, sc1, sc2):
    mx = jnp.maximum(sc1, sc2)
    e1 = jnp.exp(sc1 - mx)
    e2 = jnp.exp(sc2 - mx)
    inv = 1.0 / (e1 + e2)
    return (e1 * inv) * o1 + (e2 * inv) * o2


def _tables_body(o_a, o_p1, o_p2, sc1, sc2,
                 wx1, bx1, wd1, bd1, wx2, bx2, wd2, bd2,
                 xe_ap, ad_ap, xe_pp, ad_pp):
    hp = _combine(o_p1[...], o_p2[...], sc1[...], sc2[...])
    hp = jnp.maximum(hp, 0.0)
    ha = o_a[...]
    xe_ap[...] = jnp.dot(ha, wx1[...], preferred_element_type=jnp.float32) + bx1[...]
    ad_ap[...] = jnp.dot(hp, wd1[...], preferred_element_type=jnp.float32) + bd1[...]
    xe_pp[...] = jnp.dot(hp, wx2[...], preferred_element_type=jnp.float32) + bx2[...]
    ad_pp[...] = jnp.dot(hp, wd2[...], preferred_element_type=jnp.float32) + bd2[...]


def _final_body(aap0, aap1, app0, app1, s1r, s2r, kw, kb, q, lw, lb, out):
    s1 = s1r[...]
    s2 = s2r[...]
    o_p1 = _norm(aap0, aap1, s1, s2)
    o_p2 = _norm(app0, app1, s1, s2)
    t1 = jnp.tanh(jnp.dot(o_p1, kw[...], preferred_element_type=jnp.float32)
                  + kb[...])
    t2 = jnp.tanh(jnp.dot(o_p2, kw[...], preferred_element_type=jnp.float32)
                  + kb[...])
    m1 = jnp.sum(t1, axis=0, keepdims=True) * (1.0 / N)
    m2 = jnp.sum(t2, axis=0, keepdims=True) * (1.0 / N)
    sc1 = jnp.sum(q[...] * m1, axis=1, keepdims=True)
    sc2 = jnp.sum(q[...] * m2, axis=1, keepdims=True)
    hp = _combine(o_p1, o_p2, sc1, sc2)
    out[...] = jnp.dot(hp, lw[...], preferred_element_type=jnp.float32) + lb[...]


def _sds(shape):
    return jax.ShapeDtypeStruct(shape, jnp.float32)


_prep1 = pl.pallas_call(
    _prep1_body,
    out_shape=[_sds((N, XW))] * 6,
)

_norm3 = pl.pallas_call(
    _norm3_body,
    out_shape=[_sds((N, HIDDEN))] * 3,
)

_score2 = pl.pallas_call(
    _score2_body,
    out_shape=[_sds((1, 1))] * 2,
)

_tables = pl.pallas_call(
    _tables_body,
    out_shape=[_sds((N, XW))] * 4,
)

_final = pl.pallas_call(
    _final_body,
    out_shape=_sds((N, NUM_CLASSES)),
)


def _perm64():
    i = jnp.arange(HIDDEN)
    tgt = (i % DH) * HEADS + i // DH
    return jnp.zeros((HIDDEN, HIDDEN), jnp.float32).at[i, tgt].set(1.0)


def _mk_g(a_s):
    eye_h = jnp.eye(HEADS, dtype=jnp.float32)
    blk = (a_s[:, :, None] * eye_h[:, None, :]).reshape(HIDDEN, HEADS)
    return jnp.concatenate(
        [_perm64(), blk, blk,
         jnp.zeros((HIDDEN, XW - HIDDEN - 2 * HEADS), jnp.float32)], axis=1)


def _mk_ad(a_d):
    eye_h = jnp.eye(HEADS, dtype=jnp.float32)
    blk = (a_d[:, :, None] * eye_h[:, None, :]).reshape(HIDDEN, HEADS)
    return jnp.concatenate(
        [blk, blk, jnp.zeros((HIDDEN, XW - 2 * HEADS), jnp.float32)], axis=1)


def _fold(pw, pb, sel):
    return pw @ sel, (pb @ sel)[None, :]


def _prep_ei(ei):
    pad = EPAD - E
    r = jnp.concatenate([ei[0].astype(jnp.int32),
                         jnp.zeros((pad,), jnp.int32)])
    c = jnp.concatenate([ei[1].astype(jnp.int32),
                         jnp.full((pad,), N, jnp.int32)])
    return r.reshape(NW * NCHUNK, CHUNK), c.reshape(NW * NCHUNK, CHUNK)


def kernel(x_paper, x_author, ei_pa, ei_ap, ei_pp,
           l1_pw_p, l1_pb_p, l1_pw_a, l1_pb_a,
           l1_as_pa, l1_ad_pa, l1_as_ap, l1_ad_ap, l1_as_pp, l1_ad_pp,
           l1_q, l1_kw, l1_kb,
           l2_pw_p, l2_pb_p, l2_pw_a, l2_pb_a,
           l2_as_pa, l2_ad_pa, l2_as_ap, l2_ad_ap, l2_as_pp, l2_ad_pp,
           l2_q, l2_kw, l2_kb,
           lin_w, lin_b):
    f32 = jnp.float32
    s1 = jnp.concatenate([_perm64().T,
                          jnp.zeros((XW - HIDDEN, HIDDEN), f32)], axis=0)
    s2 = jnp.concatenate([jnp.zeros((HIDDEN, HIDDEN), f32),
                          jnp.repeat(jnp.eye(HEADS, dtype=f32), DH, axis=1),
                          jnp.zeros((XW - HIDDEN - HEADS, HIDDEN), f32)],
                         axis=0)

    wxe_pa, bxe_pa = _fold(l1_pw_p, l1_pb_p, _mk_g(l1_as_pa))
    wad_pa, bad_pa = _fold(l1_pw_a, l1_pb_a, _mk_ad(l1_ad_pa))
    wxe_ap, bxe_ap = _fold(l1_pw_a, l1_pb_a, _mk_g(l1_as_ap))
    wad_ap, bad_ap = _fold(l1_pw_p, l1_pb_p, _mk_ad(l1_ad_ap))
    wxe_pp, bxe_pp = _fold(l1_pw_p, l1_pb_p, _mk_g(l1_as_pp))
    wad_pp, bad_pp = _fold(l1_pw_p, l1_pb_p, _mk_ad(l1_ad_pp))

    xe_pa, ad_pa, xe_ap, ad_pa_dst, xe_pp, ad_pp = _prep1(
        x_paper, x_author,
        wxe_pa, bxe_pa, wad_pa, bad_pa, wxe_ap, bxe_ap,
        wad_ap, bad_ap, wxe_pp, bxe_pp, wad_pp, bad_pp)
    adst_pa, adst_ap, adst_pp = ad_pa, ad_pa_dst, ad_pp

    rpa, cpa = _prep_ei(ei_pa)
    rap, cap = _prep_ei(ei_ap)
    rpp, cpp = _prep_ei(ei_pp)

    _conv = _get_conv()
    acc_pa = _conv(xe_pa, adst_pa, rpa, cpa)
    acc_ap = _conv(xe_ap, adst_ap, rap, cap)
    acc_pp = _conv(xe_pp, adst_pp, rpp, cpp)

    wx_ap2, bx_ap2 = _fold(l2_pw_a, l2_pb_a, _mk_g(l2_as_ap))
    wd_ap2, bd_ap2 = _fold(l2_pw_p, l2_pb_p, _mk_ad(l2_ad_ap))
    wx_pp2, bx_pp2 = _fold(l2_pw_p, l2_pb_p, _mk_g(l2_as_pp))
    wd_pp2, bd_pp2 = _fold(l2_pw_p, l2_pb_p, _mk_ad(l2_ad_pp))

    o_a, o_p1, o_p2 = _norm3(
        acc_pa[0, :N], acc_pa[1, :N], acc_ap[0, :N], acc_ap[1, :N],
        acc_pp[0, :N], acc_pp[1, :N], s1, s2)
    sc1, sc2 = _score2(o_p1, o_p2, l1_kw, l1_kb[None, :], l1_q[None, :])
    xe_ap2, ad_ap2, xe_pp2, ad_pp2 = _tables(
        o_a, o_p1, o_p2, sc1, sc2,
        wx_ap2, bx_ap2, wd_ap2, bd_ap2, wx_pp2, bx_pp2, wd_pp2, bd_pp2)

    acc2_ap = _conv(xe_ap2, ad_ap2, rap, cap)
    acc2_pp = _conv(xe_pp2, ad_pp2, rpp, cpp)

    return _final(
        acc2_ap[0, :N], acc2_ap[1, :N], acc2_pp[0, :N], acc2_pp[1, :N],
        s1, s2, l2_kw, l2_kb[None, :], l2_q[None, :], lin_w, lin_b[None, :])


# speedup vs baseline: 39.0723x; 1.0583x over previous
import functools

import jax
import jax.numpy as jnp
from jax import lax
from jax.experimental import pallas as pl
from jax.experimental.pallas import tpu as pltpu
from jax.experimental.pallas import tpu_sc as plsc

N = 10000
E = 320000
IN_DIM = 128
HIDDEN = 64
HEADS = 8
DH = 8
NUM_CLASSES = 8

XW = 128
NW = 32
CHUNK = 128
NCHUNK = 80
EPT = CHUNK * NCHUNK
EPAD = EPT * NW
NPAD = 10112
RPT = NPAD // 16
HSLAB = NCHUNK // 2


def _c16(v):
    return jnp.full((16,), v, jnp.int32)


def _conv_body(xext, adst, ridx, cidx, out, idxr_v, idxc_v, xbuf0, dbuf0,
               accum):
    cid = lax.axis_index("c")
    sid = lax.axis_index("s")
    wid = sid * 2 + cid

    zero16 = jnp.zeros((16,), jnp.float32)

    def zrow(r, carry):
        for c in range(XW // 16):
            xbuf0[r, pl.ds(c * 16, 16)] = zero16
        return carry

    lax.fori_loop(0, CHUNK, zrow, 0)
    base = sid * RPT

    def zstrip(t, carry):
        pltpu.sync_copy(xbuf0.at[pl.ds(0, 8)], accum.at[pl.ds(base + t * 8, 8)])
        return carry

    lax.fori_loop(0, RPT // 8, zstrip, 0)

    plsc.subcore_barrier()

    def _compute(j, xb, db):
        def grp(g, c2):
            for k in range(16):
                e = g * 16 + k
                adv = db[e, pl.ds(0, 16)]
                asv = xb[e, pl.ds(HIDDEN, 16)]
                al = asv + adv
                al = jnp.maximum(al, al * 0.2)
                w = jnp.exp(al)
                xb[e, pl.ds(HIDDEN, 16)] = w
                for jj in range(HIDDEN // 16):
                    xv = xb[e, pl.ds(16 * jj, 16)]
                    xb[e, pl.ds(16 * jj, 16)] = xv * w
            return c2

        lax.fori_loop(0, CHUNK // 16, grp, 0)
        pltpu.sync_copy(xb, accum.at[idxc_v.at[j]], add=True)

    def chunk(j, carry):
        pltpu.sync_copy(xext.at[idxr_v.at[j]], xbuf0)
        pltpu.sync_copy(adst.at[idxc_v.at[j]], dbuf0)
        _compute(j, xbuf0, dbuf0)
        return carry

    for h in range(2):
        pltpu.sync_copy(ridx.at[pl.ds((wid * 2 + h) * HSLAB, HSLAB)], idxr_v)
        pltpu.sync_copy(cidx.at[pl.ds((wid * 2 + h) * HSLAB, HSLAB)], idxc_v)
        lax.fori_loop(0, HSLAB, chunk, 0)
    plsc.subcore_barrier()

    def dstrip(t, carry):
        pltpu.sync_copy(accum.at[pl.ds(base + t * 8, 8)], xbuf0.at[pl.ds(0, 8)])
        pltpu.sync_copy(xbuf0.at[pl.ds(0, 8)],
                        out.at[cid, pl.ds(base + t * 8, 8)])
        return carry

    lax.fori_loop(0, RPT // 8, dstrip, 0)


@functools.cache
def _get_conv():
    return pl.kernel(
        _conv_body,
        out_type=jax.ShapeDtypeStruct((2, NPAD, XW), jnp.float32),
        mesh=plsc.VectorSubcoreMesh(core_axis_name="c", subcore_axis_name="s",
                                    num_cores=2, num_subcores=16),
        compiler_params=pltpu.CompilerParams(needs_layout_passes=False),
        scratch_types=[
            pltpu.VMEM((HSLAB, CHUNK), jnp.int32),
            pltpu.VMEM((HSLAB, CHUNK), jnp.int32),
            pltpu.VMEM((CHUNK, XW), jnp.float32),
            pltpu.VMEM((CHUNK, XW), jnp.float32),
            pltpu.VMEM_SHARED((NPAD, XW), jnp.float32),
        ],
    )


def _prep1_body(xp, xa, w1, b1, w2, b2, w3, b3, w4, b4, w5, b5, w6, b6,
                o1, o2, o3, o4, o5, o6):
    xpv = xp[...]
    xav = xa[...]
    o1[...] = jnp.dot(xpv, w1[...], preferred_element_type=jnp.float32) + b1[...]
    o2[...] = jnp.dot(xav, w2[...], preferred_element_type=jnp.float32) + b2[...]
    o3[...] = jnp.dot(xav, w3[...], preferred_element_type=jnp.float32) + b3[...]
    o4[...] = jnp.dot(xpv, w4[...], preferred_element_type=jnp.float32) + b4[...]
    o5[...] = jnp.dot(xpv, w5[...], preferred_element_type=jnp.float32) + b5[...]
    o6[...] = jnp.dot(xpv, w6[...], preferred_element_type=jnp.float32) + b6[...]


def _norm(acc0, acc1, s1, s2):
    acc = acc0[...] + acc1[...]
    num = jnp.dot(acc, s1, preferred_element_type=jnp.float32)
    den = jnp.dot(acc, s2, preferred_element_type=jnp.float32)
    return jnp.maximum(num / (den + 1e-16), 0.0)


def _sem_attn(o1, o2, kw, kb, q):
    t1 = jnp.tanh(jnp.dot(o1, kw, preferred_element_type=jnp.float32) + kb)
    t2 = jnp.tanh(jnp.dot(o2, kw, preferred_element_type=jnp.float32) + kb)
    m1 = jnp.sum(t1, axis=0, keepdims=True) * (1.0 / N)
    m2 = jnp.sum(t2, axis=0, keepdims=True) * (1.0 / N)
    sc1 = jnp.sum(q * m1, axis=1, keepdims=True)
    sc2 = jnp.sum(q * m2, axis=1, keepdims=True)
    mx = jnp.maximum(sc1, sc2)
    e1 = jnp.exp(sc1 - mx)
    e2 = jnp.exp(sc2 - mx)
    inv = 1.0 / (e1 + e2)
    return (e1 * inv) * o1 + (e2 * inv) * o2


def _norm3_body(apa0, apa1, aap0, aap1, app0, app1, s1r, s2r,
                o_a_ref, o_p1_ref, o_p2_ref):
    s1 = s1r[...]
    s2 = s2r[...]
    o_a_ref[...] = _norm(apa0, apa1, s1, s2)
    o_p1_ref[...] = _norm(aap0, aap1, s1, s2)
    o_p2_ref[...] = _norm(app0, app1, s1, s2)


def _score2_body(o1, o2, kw, kb, q, sc1_ref, sc2_ref):
    t1 = jnp.tanh(jnp.dot(o1[...], kw[...], preferred_element_type=jnp.float32)
                  + kb[...])
    t2 = jnp.tanh(jnp.dot(o2[...], kw[...], preferred_element_type=jnp.float32)
                  + kb[...])
    m1 = jnp.sum(t1, axis=0, keepdims=True) * (1.0 / N)
    m2 = jnp.sum(t2, axis=0, keepdims=True) * (1.0 / N)
    sc1_ref[...] = jnp.sum(q[...] * m1, axis=1, keepdims=True)
    sc2_ref[...] = jnp.sum(q[...] * m2, axis=1, keepdims=True)


def _combine(o1, o2, sc1, sc2):
    mx = jnp.maximum(sc1, sc2)
    e1 = jnp.exp(sc1 - mx)
    e2 = jnp.exp(sc2 - mx)
    inv = 1.0 / (e1 + e2)
    return (e1 * inv) * o1 + (e2 * inv) * o2


def _tables_body(o_a, o_p1, o_p2, sc1, sc2,
                 wx1, bx1, wd1, bd1, wx2, bx2, wd2, bd2,
                 xe_ap, ad_ap, xe_pp, ad_pp):
    hp = _combine(o_p1[...], o_p2[...], sc1[...], sc2[...])
    hp = jnp.maximum(hp, 0.0)
    ha = o_a[...]
    xe_ap[...] = jnp.dot(ha, wx1[...], preferred_element_type=jnp.float32) + bx1[...]
    ad_ap[...] = jnp.dot(hp, wd1[...], preferred_element_type=jnp.float32) + bd1[...]
    xe_pp[...] = jnp.dot(hp, wx2[...], preferred_element_type=jnp.float32) + bx2[...]
    ad_pp[...] = jnp.dot(hp, wd2[...], preferred_element_type=jnp.float32) + bd2[...]


def _final_body(aap0, aap1, app0, app1, s1r, s2r, kw, kb, q, lw, lb, out):
    s1 = s1r[...]
    s2 = s2r[...]
    o_p1 = _norm(aap0, aap1, s1, s2)
    o_p2 = _norm(app0, app1, s1, s2)
    t1 = jnp.tanh(jnp.dot(o_p1, kw[...], preferred_element_type=jnp.float32)
                  + kb[...])
    t2 = jnp.tanh(jnp.dot(o_p2, kw[...], preferred_element_type=jnp.float32)
                  + kb[...])
    m1 = jnp.sum(t1, axis=0, keepdims=True) * (1.0 / N)
    m2 = jnp.sum(t2, axis=0, keepdims=True) * (1.0 / N)
    sc1 = jnp.sum(q[...] * m1, axis=1, keepdims=True)
    sc2 = jnp.sum(q[...] * m2, axis=1, keepdims=True)
    hp = _combine(o_p1, o_p2, sc1, sc2)
    out[...] = jnp.dot(hp, lw[...], preferred_element_type=jnp.float32) + lb[...]


def _sds(shape):
    return jax.ShapeDtypeStruct(shape, jnp.float32)


_prep1 = pl.pallas_call(
    _prep1_body,
    out_shape=[_sds((N, XW))] * 6,
)

_norm3 = pl.pallas_call(
    _norm3_body,
    out_shape=[_sds((N, HIDDEN))] * 3,
)

_score2 = pl.pallas_call(
    _score2_body,
    out_shape=[_sds((1, 1))] * 2,
)

_tables = pl.pallas_call(
    _tables_body,
    out_shape=[_sds((N, XW))] * 4,
)

_final = pl.pallas_call(
    _final_body,
    out_shape=_sds((N, NUM_CLASSES)),
)


def _perm64():
    i = jnp.arange(HIDDEN)
    tgt = (i % DH) * HEADS + i // DH
    return jnp.zeros((HIDDEN, HIDDEN), jnp.float32).at[i, tgt].set(1.0)


def _mk_g(a_s):
    eye_h = jnp.eye(HEADS, dtype=jnp.float32)
    blk = (a_s[:, :, None] * eye_h[:, None, :]).reshape(HIDDEN, HEADS)
    return jnp.concatenate(
        [_perm64(), blk, blk,
         jnp.zeros((HIDDEN, XW - HIDDEN - 2 * HEADS), jnp.float32)], axis=1)


def _mk_ad(a_d):
    eye_h = jnp.eye(HEADS, dtype=jnp.float32)
    blk = (a_d[:, :, None] * eye_h[:, None, :]).reshape(HIDDEN, HEADS)
    return jnp.concatenate(
        [blk, blk, jnp.zeros((HIDDEN, XW - 2 * HEADS), jnp.float32)], axis=1)


def _fold(pw, pb, sel):
    return pw @ sel, (pb @ sel)[None, :]


def _prep_ei(ei):
    pad = EPAD - E
    r = jnp.concatenate([ei[0].astype(jnp.int32),
                         jnp.zeros((pad,), jnp.int32)])
    c = jnp.concatenate([ei[1].astype(jnp.int32),
                         jnp.full((pad,), N, jnp.int32)])
    return r.reshape(NW * NCHUNK, CHUNK), c.reshape(NW * NCHUNK, CHUNK)


def kernel(x_paper, x_author, ei_pa, ei_ap, ei_pp,
           l1_pw_p, l1_pb_p, l1_pw_a, l1_pb_a,
           l1_as_pa, l1_ad_pa, l1_as_ap, l1_ad_ap, l1_as_pp, l1_ad_pp,
           l1_q, l1_kw, l1_kb,
           l2_pw_p, l2_pb_p, l2_pw_a, l2_pb_a,
           l2_as_pa, l2_ad_pa, l2_as_ap, l2_ad_ap, l2_as_pp, l2_ad_pp,
           l2_q, l2_kw, l2_kb,
           lin_w, lin_b):
    f32 = jnp.float32
    s1 = jnp.concatenate([_perm64().T,
                          jnp.zeros((XW - HIDDEN, HIDDEN), f32)], axis=0)
    s2 = jnp.concatenate([jnp.zeros((HIDDEN, HIDDEN), f32),
                          jnp.repeat(jnp.eye(HEADS, dtype=f32), DH, axis=1),
                          jnp.zeros((XW - HIDDEN - HEADS, HIDDEN), f32)],
                         axis=0)

    wxe_pa, bxe_pa = _fold(l1_pw_p, l1_pb_p, _mk_g(l1_as_pa))
    wad_pa, bad_pa = _fold(l1_pw_a, l1_pb_a, _mk_ad(l1_ad_pa))
    wxe_ap, bxe_ap = _fold(l1_pw_a, l1_pb_a, _mk_g(l1_as_ap))
    wad_ap, bad_ap = _fold(l1_pw_p, l1_pb_p, _mk_ad(l1_ad_ap))
    wxe_pp, bxe_pp = _fold(l1_pw_p, l1_pb_p, _mk_g(l1_as_pp))
    wad_pp, bad_pp = _fold(l1_pw_p, l1_pb_p, _mk_ad(l1_ad_pp))

    xe_pa, ad_pa, xe_ap, ad_pa_dst, xe_pp, ad_pp = _prep1(
        x_paper, x_author,
        wxe_pa, bxe_pa, wad_pa, bad_pa, wxe_ap, bxe_ap,
        wad_ap, bad_ap, wxe_pp, bxe_pp, wad_pp, bad_pp)
    adst_pa, adst_ap, adst_pp = ad_pa, ad_pa_dst, ad_pp

    rpa, cpa = _prep_ei(ei_pa)
    rap, cap = _prep_ei(ei_ap)
    rpp, cpp = _prep_ei(ei_pp)

    _conv = _get_conv()
    acc_pa = _conv(xe_pa, adst_pa, rpa, cpa)
    acc_ap = _conv(xe_ap, adst_ap, rap, cap)
    acc_pp = _conv(xe_pp, adst_pp, rpp, cpp)

    wx_ap2, bx_ap2 = _fold(l2_pw_a, l2_pb_a, _mk_g(l2_as_ap))
    wd_ap2, bd_ap2 = _fold(l2_pw_p, l2_pb_p, _mk_ad(l2_ad_ap))
    wx_pp2, bx_pp2 = _fold(l2_pw_p, l2_pb_p, _mk_g(l2_as_pp))
    wd_pp2, bd_pp2 = _fold(l2_pw_p, l2_pb_p, _mk_ad(l2_ad_pp))

    o_a, o_p1, o_p2 = _norm3(
        acc_pa[0, :N], acc_pa[1, :N], acc_ap[0, :N], acc_ap[1, :N],
        acc_pp[0, :N], acc_pp[1, :N], s1, s2)
    sc1, sc2 = _score2(o_p1, o_p2, l1_kw, l1_kb[None, :], l1_q[None, :])
    xe_ap2, ad_ap2, xe_pp2, ad_pp2 = _tables(
        o_a, o_p1, o_p2, sc1, sc2,
        wx_ap2, bx_ap2, wd_ap2, bd_ap2, wx_pp2, bx_pp2, wd_pp2, bd_pp2)

    acc2_ap = _conv(xe_ap2, ad_ap2, rap, cap)
    acc2_pp = _conv(xe_pp2, ad_pp2, rpp, cpp)

    return _final(
        acc2_ap[0, :N], acc2_ap[1, :N], acc2_pp[0, :N], acc2_pp[1, :N],
        s1, s2, l2_kw, l2_kb[None, :], l2_q[None, :], lin_w, lin_b[None, :])
